# Initial kernel scaffold; baseline (speedup 1.0000x reference)
import jax
import jax.numpy as jnp
from jax.experimental import pallas as pl


def kernel(log_s0, s0, constraints):
    raise NotImplementedError("write your pallas kernel here")


# speedup vs baseline: 3.1138x; 3.1138x over previous
import jax
import jax.numpy as jnp
import numpy as np
from jax.experimental import pallas as pl

B = 16384
S = 16
P = 16
EPS = 1e-8
ACC_STATE = 15
BLOCK_B = 1024


def _automaton_constants():
    rng = np.random.RandomState(0)
    exists = (rng.rand(S, S) < 0.35).astype(np.float32)
    exists[np.arange(S), (np.arange(S) + 1) % S] = 1.0
    mask = (rng.rand(S, S, P) < 0.2).astype(np.float32) * exists[:, :, None]
    m2t = np.transpose(mask, (1, 0, 2)).reshape(S * S, P)
    selt = (exists.T[:, :, None] * np.eye(S, dtype=np.float32)[None, :, :]).reshape(S * S, S)
    return jnp.asarray(m2t), jnp.asarray(selt)


def _fused_kernel(c_ref, s0_ref, m2t_ref, selt_ref, ln_ref, ns_ref, acc_ref):
    bB = c_ref.shape[0]
    f32 = jnp.float32
    dn = (((1,), (1,)), ((), ()))
    lc = jnp.log(c_ref[:] + 1e-12)
    tl = jax.lax.dot_general(m2t_ref[:], lc, dn,
                             preferred_element_type=f32)
    gate = jax.lax.dot_general(selt_ref[:], s0_ref[:], dn,
                               preferred_element_type=f32)
    proofs = jnp.exp(tl) * gate
    x = proofs.reshape(S, S, bB)

    m1 = jnp.max(x, axis=1)
    c1 = jnp.sum((x >= m1[:, None, :]).astype(f32), axis=1)
    x2 = jnp.where(x < m1[:, None, :], x, -1.0)
    m2 = jnp.max(x2, axis=1)
    c2 = jnp.sum((x2 >= m2[:, None, :]).astype(f32), axis=1)
    x3 = jnp.where(x2 < m2[:, None, :], x2, -1.0)
    m3 = jnp.max(x3, axis=1)

    a1 = jnp.minimum(c1, 3.0)
    a2 = jnp.clip(3.0 - a1, 0.0, c2)
    a3 = jnp.maximum(3.0 - a1 - a2, 0.0)
    total = a1 * m1 + a2 * m2 + a3 * m3

    denom = jnp.sum(total, axis=0, keepdims=True) + EPS
    nxt = total / denom
    ln_ref[:] = jnp.log(nxt + EPS).T
    ns_ref[:] = nxt.T
    acc_ref[:] = total[ACC_STATE][:, None]


def kernel(log_s0, s0, constraints):
    del log_s0
    m2t, selt = _automaton_constants()
    grid = (B // BLOCK_B,)
    ln, ns, acc = pl.pallas_call(
        _fused_kernel,
        grid=grid,
        in_specs=[
            pl.BlockSpec((BLOCK_B, P), lambda i: (i, 0)),
            pl.BlockSpec((BLOCK_B, S), lambda i: (i, 0)),
            pl.BlockSpec((S * S, P), lambda i: (0, 0)),
            pl.BlockSpec((S * S, S), lambda i: (0, 0)),
        ],
        out_specs=[
            pl.BlockSpec((BLOCK_B, S), lambda i: (i, 0)),
            pl.BlockSpec((BLOCK_B, S), lambda i: (i, 0)),
            pl.BlockSpec((BLOCK_B, 1), lambda i: (i, 0)),
        ],
        out_shape=[
            jax.ShapeDtypeStruct((B, S), jnp.float32),
            jax.ShapeDtypeStruct((B, S), jnp.float32),
            jax.ShapeDtypeStruct((B, 1), jnp.float32),
        ],
    )(constraints, s0, m2t, selt)
    return (ln, ns, acc.reshape(B))


# speedup vs baseline: 3.5743x; 1.1479x over previous
import jax
import jax.numpy as jnp
import numpy as np
from jax.experimental import pallas as pl

B = 16384
S = 16
P = 16
EPS = 1e-8
ACC_STATE = 15
BLOCK_B = 1024


def _automaton_constants():
    rng = np.random.RandomState(0)
    exists = (rng.rand(S, S) < 0.35).astype(np.float32)
    exists[np.arange(S), (np.arange(S) + 1) % S] = 1.0
    mask = (rng.rand(S, S, P) < 0.2).astype(np.float32) * exists[:, :, None]
    m2t = mask.reshape(S * S, P)
    selt = (exists[:, :, None] * np.eye(S, dtype=np.float32)[:, None, :]).reshape(S * S, S)
    return jnp.asarray(m2t), jnp.asarray(selt)


def _fused_kernel(c_ref, s0_ref, m2t_ref, selt_ref, ln_ref, ns_ref, acc_ref):
    f32 = jnp.float32
    dn = (((1,), (1,)), ((), ()))
    lc = jnp.log(c_ref[:] + 1e-12)
    tl = jax.lax.dot_general(m2t_ref[:], lc, dn,
                             preferred_element_type=f32)
    gate = jax.lax.dot_general(selt_ref[:], s0_ref[:], dn,
                               preferred_element_type=f32)
    proofs = jnp.exp(tl) * gate

    neg = jnp.full((S, proofs.shape[1]), -1.0, dtype=f32)
    m1, m2, m3 = neg, neg, neg
    for s0i in range(S):
        v = proofs[s0i * S:(s0i + 1) * S, :]
        nm1 = jnp.maximum(m1, v)
        t = jnp.minimum(m1, v)
        nm2 = jnp.maximum(m2, t)
        t2 = jnp.minimum(m2, t)
        m3 = jnp.maximum(m3, t2)
        m1, m2 = nm1, nm2
    total = m1 + m2 + m3

    denom = jnp.sum(total, axis=0, keepdims=True) + EPS
    nxt = total / denom
    ln_ref[:] = jnp.log(nxt + EPS).T
    ns_ref[:] = nxt.T
    acc_ref[:] = total[ACC_STATE][:, None]


def kernel(log_s0, s0, constraints):
    del log_s0
    m2t, selt = _automaton_constants()
    grid = (B // BLOCK_B,)
    ln, ns, acc = pl.pallas_call(
        _fused_kernel,
        grid=grid,
        in_specs=[
            pl.BlockSpec((BLOCK_B, P), lambda i: (i, 0)),
            pl.BlockSpec((BLOCK_B, S), lambda i: (i, 0)),
            pl.BlockSpec((S * S, P), lambda i: (0, 0)),
            pl.BlockSpec((S * S, S), lambda i: (0, 0)),
        ],
        out_specs=[
            pl.BlockSpec((BLOCK_B, S), lambda i: (i, 0)),
            pl.BlockSpec((BLOCK_B, S), lambda i: (i, 0)),
            pl.BlockSpec((BLOCK_B, 1), lambda i: (i, 0)),
        ],
        out_shape=[
            jax.ShapeDtypeStruct((B, S), jnp.float32),
            jax.ShapeDtypeStruct((B, S), jnp.float32),
            jax.ShapeDtypeStruct((B, 1), jnp.float32),
        ],
    )(constraints, s0, m2t, selt)
    return (ln, ns, acc.reshape(B))


# speedup vs baseline: 4.0093x; 1.1217x over previous
import jax
import jax.numpy as jnp
import numpy as np
from jax.experimental import pallas as pl

B = 16384
S = 16
P = 16
EPS = 1e-8
ACC_STATE = 15
BLOCK_B = 4096


def _automaton_constants():
    rng = np.random.RandomState(0)
    exists = (rng.rand(S, S) < 0.35).astype(np.float32)
    exists[np.arange(S), (np.arange(S) + 1) % S] = 1.0
    mask = (rng.rand(S, S, P) < 0.2).astype(np.float32) * exists[:, :, None]
    m2t = mask.reshape(S * S, P)
    selt = (exists[:, :, None] * np.eye(S, dtype=np.float32)[:, None, :]).reshape(S * S, S)
    return jnp.asarray(m2t), jnp.asarray(selt)


def _fused_kernel(c_ref, s0_ref, m2t_ref, selt_ref, ln_ref, ns_ref, acc_ref):
    f32 = jnp.float32
    dn = (((1,), (1,)), ((), ()))
    lc = jnp.log(c_ref[:] + 1e-12)
    tl = jax.lax.dot_general(m2t_ref[:], lc, dn,
                             preferred_element_type=f32)
    gate = jax.lax.dot_general(selt_ref[:], s0_ref[:], dn,
                               preferred_element_type=f32)
    proofs = jnp.exp(tl) * gate

    neg = jnp.full((S, proofs.shape[1]), -1.0, dtype=f32)
    m1, m2, m3 = neg, neg, neg
    for s0i in range(S):
        v = proofs[s0i * S:(s0i + 1) * S, :]
        nm1 = jnp.maximum(m1, v)
        t = jnp.minimum(m1, v)
        nm2 = jnp.maximum(m2, t)
        t2 = jnp.minimum(m2, t)
        m3 = jnp.maximum(m3, t2)
        m1, m2 = nm1, nm2
    total = m1 + m2 + m3

    denom = jnp.sum(total, axis=0, keepdims=True) + EPS
    nxt = total / denom
    ln_ref[:] = jnp.log(nxt + EPS).T
    ns_ref[:] = nxt.T
    acc_ref[:] = total[ACC_STATE][:, None]


def kernel(log_s0, s0, constraints):
    del log_s0
    m2t, selt = _automaton_constants()
    grid = (B // BLOCK_B,)
    ln, ns, acc = pl.pallas_call(
        _fused_kernel,
        grid=grid,
        in_specs=[
            pl.BlockSpec((BLOCK_B, P), lambda i: (i, 0)),
            pl.BlockSpec((BLOCK_B, S), lambda i: (i, 0)),
            pl.BlockSpec((S * S, P), lambda i: (0, 0)),
            pl.BlockSpec((S * S, S), lambda i: (0, 0)),
        ],
        out_specs=[
            pl.BlockSpec((BLOCK_B, S), lambda i: (i, 0)),
            pl.BlockSpec((BLOCK_B, S), lambda i: (i, 0)),
            pl.BlockSpec((BLOCK_B, 1), lambda i: (i, 0)),
        ],
        out_shape=[
            jax.ShapeDtypeStruct((B, S), jnp.float32),
            jax.ShapeDtypeStruct((B, S), jnp.float32),
            jax.ShapeDtypeStruct((B, 1), jnp.float32),
        ],
    )(constraints, s0, m2t, selt)
    return (ln, ns, acc.reshape(B))
